# baseline (speedup 1.0000x reference)
import jax
import jax.numpy as jnp
from jax import lax
from jax.experimental import pallas as pl

N = 16384
C = 1000
NB = 16
W = N // NB
BR = 128
GRID = N // BR
S = 128
SEARCH_ITERS = 31


def _phase_a_kernel(x_ref, lbl_ref, conf_ref, acc_ref):
    x = x_ref[...]
    m = jnp.max(x, axis=1, keepdims=True)
    col = lax.broadcasted_iota(jnp.int32, x.shape, 1)
    big = jnp.int32(2 ** 30)
    pidx = jnp.min(jnp.where(x == m, col, big), axis=1, keepdims=True)
    conf_ref[...] = m
    acc_ref[...] = (pidx == lbl_ref[...]).astype(jnp.float32)


def _phase_b_kernel(conf_ref, acc_ref, ece_ref, ys_ref):
    conf = conf_ref[...]
    acc = acc_ref[...]
    u = lax.bitcast_convert_type(conf, jnp.int32)

    ranks = [b * W for b in range(1, NB)]

    def search_body(_, carry):
        los, his = carry
        nlos, nhis = [], []
        for b in range(NB - 1):
            lo, hi = los[b], his[b]
            mid = lax.div(lo + hi, jnp.int32(2))
            cnt = jnp.sum((u <= mid).astype(jnp.float32))
            pred = cnt >= jnp.float32(ranks[b] + 1)
            nhis.append(jnp.where(pred, mid, hi))
            nlos.append(jnp.where(pred, lo, mid + jnp.int32(1)))
        return tuple(nlos), tuple(nhis)

    init = (tuple(jnp.int32(0) for _ in range(NB - 1)),
            tuple(jnp.int32(0x3F800000) for _ in range(NB - 1)))
    los, _ = lax.fori_loop(0, SEARCH_ITERS, search_body, init)

    r_iota = lax.broadcasted_iota(jnp.int32, (S, S), 0)
    c_iota = lax.broadcasted_iota(jnp.int32, (S, S), 1)
    l_strict = (c_iota < r_iota).astype(jnp.float32)
    u_strict = (r_iota < c_iota).astype(jnp.float32)
    ones_mat = jnp.ones((S, S), jnp.float32)

    binf = jnp.zeros((S, S), jnp.float32)
    for b in range(NB - 1):
        v = los[b]
        n_low = jnp.float32(ranks[b]) - jnp.sum((u < v).astype(jnp.float32))
        mb = (u == v).astype(jnp.float32)
        t1 = jnp.dot(mb, u_strict, preferred_element_type=jnp.float32)
        rowtot = jnp.dot(mb, ones_mat, preferred_element_type=jnp.float32)
        t2 = jnp.dot(l_strict, rowtot, preferred_element_type=jnp.float32)
        tier = t1 + t2
        above = (jnp.where(u > v, 1.0, 0.0)
                 + jnp.where((u == v) & (tier >= n_low), 1.0, 0.0))
        binf = binf + above

    lane16 = lax.broadcasted_iota(jnp.int32, (1, NB), 1)
    ece = jnp.float32(0.0)
    ys_row = jnp.zeros((1, NB), jnp.float32)
    inv_w = jnp.float32(1.0 / W)
    for k in range(NB):
        mk = (binf == jnp.float32(k)).astype(jnp.float32)
        sc = jnp.sum(conf * mk) * inv_w
        sa = jnp.sum(acc * mk) * inv_w
        ece = ece + jnp.abs(sc - sa)
        ys_row = ys_row + jnp.where(lane16 == k, sa, 0.0)
    ece_ref[...] = jnp.broadcast_to(ece * jnp.float32(float(W) / float(N)),
                                    (1, 1))
    ys_ref[...] = ys_row


@jax.jit
def kernel(softmaxes, labels):
    lbl2 = labels.astype(jnp.int32).reshape(N, 1)
    conf, accv = pl.pallas_call(
        _phase_a_kernel,
        grid=(GRID,),
        in_specs=[pl.BlockSpec((BR, C), lambda i: (i, 0)),
                  pl.BlockSpec((BR, 1), lambda i: (i, 0))],
        out_specs=[pl.BlockSpec((BR, 1), lambda i: (i, 0)),
                   pl.BlockSpec((BR, 1), lambda i: (i, 0))],
        out_shape=[jax.ShapeDtypeStruct((N, 1), jnp.float32),
                   jax.ShapeDtypeStruct((N, 1), jnp.float32)],
    )(softmaxes, lbl2)

    ece, ys = pl.pallas_call(
        _phase_b_kernel,
        in_specs=[pl.BlockSpec((S, S), lambda: (0, 0)),
                  pl.BlockSpec((S, S), lambda: (0, 0))],
        out_specs=[pl.BlockSpec((1, 1), lambda: (0, 0)),
                   pl.BlockSpec((1, NB), lambda: (0, 0))],
        out_shape=[jax.ShapeDtypeStruct((1, 1), jnp.float32),
                   jax.ShapeDtypeStruct((1, NB), jnp.float32)],
    )(conf.reshape(S, S), accv.reshape(S, S))
    return (ece.reshape(1), ys.reshape(NB))
